# baseline (speedup 1.0000x reference)
import jax
import jax.numpy as jnp
from jax.experimental import pallas as pl
from jax.experimental.pallas import tpu as pltpu

ROWS_PER_BLOCK = 8


def _body(ts_ref, acp_ref, x_ref, n_ref, o_ref):
    i = pl.program_id(0)
    base = i * ROWS_PER_BLOCK
    for r in range(ROWS_PER_BLOCK):
        t = ts_ref[base + r]
        a = acp_ref[t]
        sa = jnp.sqrt(a)
        sb = jnp.sqrt(1.0 - a)
        o_ref[r] = sa * x_ref[r] + sb * n_ref[r]


def kernel(original_samples, noise, timesteps, alphas_cumprod):
    b, c, h, w = original_samples.shape
    ts = timesteps.astype(jnp.int32)

    grid = (b // ROWS_PER_BLOCK,)
    blk = pl.BlockSpec((ROWS_PER_BLOCK, c, h, w), lambda i: (i, 0, 0, 0))
    out = pl.pallas_call(
        _body,
        grid=grid,
        in_specs=[
            pl.BlockSpec(memory_space=pltpu.SMEM),
            pl.BlockSpec(memory_space=pltpu.SMEM),
            blk,
            blk,
        ],
        out_specs=blk,
        out_shape=jax.ShapeDtypeStruct((b, c, h, w), jnp.float32),
        compiler_params=pltpu.CompilerParams(
            dimension_semantics=("arbitrary",),
        ),
    )(ts, alphas_cumprod, original_samples, noise)
    return out


# speedup vs baseline: 1.5809x; 1.5809x over previous
import jax
import jax.numpy as jnp
from jax.experimental import pallas as pl
from jax.experimental.pallas import tpu as pltpu

ROWS_PER_BLOCK = 8


def _body(ts_ref, acp_ref, x_ref, n_ref, o_ref):
    i = pl.program_id(0)
    base = i * ROWS_PER_BLOCK
    for r in range(ROWS_PER_BLOCK):
        t = ts_ref[base + r]
        a = acp_ref[t]
        sa = jnp.sqrt(a)
        sb = jnp.sqrt(1.0 - a)
        o_ref[r] = sa * x_ref[r] + sb * n_ref[r]


def kernel(original_samples, noise, timesteps, alphas_cumprod):
    b = original_samples.shape[0]
    x = original_samples.reshape(b, 128, 128)
    n = noise.reshape(b, 128, 128)
    ts = timesteps.astype(jnp.int32)

    grid = (b // ROWS_PER_BLOCK,)
    blk = pl.BlockSpec((ROWS_PER_BLOCK, 128, 128), lambda i: (i, 0, 0))
    out = pl.pallas_call(
        _body,
        grid=grid,
        in_specs=[
            pl.BlockSpec(memory_space=pltpu.SMEM),
            pl.BlockSpec(memory_space=pltpu.SMEM),
            blk,
            blk,
        ],
        out_specs=blk,
        out_shape=jax.ShapeDtypeStruct((b, 128, 128), jnp.float32),
        compiler_params=pltpu.CompilerParams(
            dimension_semantics=("arbitrary",),
        ),
    )(ts, alphas_cumprod, x, n)
    return out.reshape(original_samples.shape)


# speedup vs baseline: 6.3029x; 3.9869x over previous
import jax
import jax.numpy as jnp
from jax.experimental import pallas as pl
from jax.experimental.pallas import tpu as pltpu

_BS = 4096


def _body(ts_ref, tab_ref, x_ref, n_ref, o_ref, s_ref):
    i = pl.program_id(0)

    @pl.when(i == 0)
    def _():
        idx = jax.lax.broadcasted_iota(jnp.int32, (1024, 256), 0)
        m = jnp.where(idx == ts_ref[0:1, :], 1.0, 0.0)
        a = jnp.sum(m * tab_ref[...], axis=0)
        s_ref[0:1, :] = jnp.sqrt(a)[None, :]
        s_ref[1:2, :] = jnp.sqrt(1.0 - a)[None, :]

    o_ref[...] = s_ref[0:1, :] * x_ref[...] + s_ref[1:2, :] * n_ref[...]


def kernel(original_samples, noise, timesteps, alphas_cumprod):
    b, c, h, w = original_samples.shape
    p = c * h * w
    xt = original_samples.transpose(1, 2, 3, 0).reshape(p, b)
    nt = noise.transpose(1, 2, 3, 0).reshape(p, b)
    ts2 = timesteps.astype(jnp.int32).reshape(1, b)
    tab = jnp.pad(alphas_cumprod, (0, 1024 - alphas_cumprod.shape[0])).reshape(1024, 1)

    blk = pl.BlockSpec((_BS, b), lambda i: (i, 0))
    small = lambda shape: pl.BlockSpec(shape, lambda i: (0, 0))
    out = pl.pallas_call(
        _body,
        grid=(p // _BS,),
        in_specs=[small((1, b)), small((1024, 1)), blk, blk],
        out_specs=blk,
        out_shape=jax.ShapeDtypeStruct((p, b), jnp.float32),
        scratch_shapes=[pltpu.VMEM((8, b), jnp.float32)],
        compiler_params=pltpu.CompilerParams(dimension_semantics=("arbitrary",)),
    )(ts2, tab, xt, nt)
    return out.reshape(c, h, w, b).transpose(3, 0, 1, 2)
